# baseline (speedup 1.0000x reference)
import functools

import jax
import jax.numpy as jnp
from jax import lax
from jax.experimental import pallas as pl
from jax.experimental.pallas import tpu as pltpu
from jax.experimental.pallas import tpu_sc as plsc

N = 10000
E = 320000
D = 128
H = 64
G = 64
OUT = 2

C = 125
NW = 32
CHW = E // (NW * C)
NP = 10240
RPS = NP // 16
DEGW = 16
NSLOT = 8

_mesh = plsc.VectorSubcoreMesh(core_axis_name="c", subcore_axis_name="s")
_sc_params = pltpu.CompilerParams(use_tc_tiling_on_sc=False)


@functools.partial(
    pl.kernel,
    out_type=jax.ShapeDtypeStruct((2, NP, DEGW), jnp.float32),
    mesh=_mesh,
    scratch_types=[
        pltpu.VMEM((CHW, C), jnp.int32),
        pltpu.VMEM((C, DEGW), jnp.float32),
        pltpu.VMEM_SHARED((NP, DEGW), jnp.float32),
        [pltpu.SemaphoreType.DMA] * 4,
    ],
    compiler_params=_sc_params,
)
def _sc_degree(dst_hbm, ones_hbm, zeros_hbm, out_hbm, dst_v, ones_v, acc, sems):
    cid = lax.axis_index("c")
    sid = lax.axis_index("s")
    wid = sid * 2 + cid
    pltpu.sync_copy(zeros_hbm.at[pl.ds(sid * RPS, RPS)],
                    acc.at[pl.ds(sid * RPS, RPS)])
    pltpu.sync_copy(dst_hbm.at[wid], dst_v)
    pltpu.sync_copy(ones_hbm, ones_v)
    plsc.subcore_barrier()

    def _scat(j, b):
        pltpu.async_copy(ones_v, acc.at[dst_v.at[j]], sems[b], add=True)

    def _drain(j, b):
        pltpu.make_async_copy(ones_v, acc.at[dst_v.at[j]], sems[b]).wait()

    for b in range(4):
        _scat(b, b)

    def body(g, carry):
        for b in range(4):
            j = 4 + g * 4 + b
            _drain(j, b)
            _scat(j, b)
        return carry

    lax.fori_loop(0, (CHW - 4) // 4, body, 0)
    for b in range(4):
        _drain(0, b)
    plsc.subcore_barrier()
    pltpu.sync_copy(acc.at[pl.ds(sid * RPS, RPS)],
                    out_hbm.at[cid, pl.ds(sid * RPS, RPS)])


def _make_sc_scatter(dtype):
  @functools.partial(
      pl.kernel,
      out_type=jax.ShapeDtypeStruct((4, NP, H), dtype),
      mesh=_mesh,
      scratch_types=[
          pltpu.VMEM((CHW, C), jnp.int32),
          pltpu.VMEM((CHW, C), jnp.int32),
          pltpu.VMEM((NSLOT, C, H), dtype),
          pltpu.VMEM_SHARED((NP, H), dtype),
          pltpu.VMEM_SHARED((NP, H), dtype),
          [pltpu.SemaphoreType.DMA] * NSLOT,
          [pltpu.SemaphoreType.DMA] * NSLOT,
      ],
      compiler_params=_sc_params,
  )
  def _sc_scatter(h_hbm, src_hbm, dst_hbm, zeros_hbm, out_hbm,
                  src_v, dst_v, rows_v, acc_e, acc_o, gsems, ssems):
    cid = lax.axis_index("c")
    sid = lax.axis_index("s")
    wid = sid * 2 + cid
    pltpu.sync_copy(zeros_hbm.at[pl.ds(sid * RPS, RPS)],
                    acc_e.at[pl.ds(sid * RPS, RPS)])
    pltpu.sync_copy(zeros_hbm.at[pl.ds(sid * RPS, RPS)],
                    acc_o.at[pl.ds(sid * RPS, RPS)])
    pltpu.sync_copy(src_hbm.at[wid], src_v)
    pltpu.sync_copy(dst_hbm.at[wid], dst_v)
    plsc.subcore_barrier()

    def _acc(j):
        return acc_e if j % 2 == 0 else acc_o

    def _gather(j, b):
        pltpu.async_copy(h_hbm.at[src_v.at[j]], rows_v.at[b], gsems[b])

    def _gwait(j, b):
        pltpu.make_async_copy(h_hbm.at[src_v.at[j]], rows_v.at[b],
                              gsems[b]).wait()

    def _scat(j, b, par):
        pltpu.async_copy(rows_v.at[b], _acc(par).at[dst_v.at[j]], ssems[b],
                         add=True)

    def _swait(j, b, par):
        pltpu.make_async_copy(rows_v.at[b], _acc(par).at[dst_v.at[j]],
                              ssems[b]).wait()

    for b in range(4):
        _gather(b, b)
    for j in range(4):
        _gwait(j, j)
        _scat(j, j, j)
        _gather(j + 4, j + 4)

    def body(g, carry):
        for boff in range(8):
            j = 4 + g * 8 + boff
            b = (4 + boff) % NSLOT
            b4 = boff % NSLOT
            _gwait(j, b)
            _scat(j, b, boff)
            _swait(j - 4, b4, boff)
            _gather(j + 4, b4)
        return carry

    lax.fori_loop(0, (CHW - 8) // 8, body, 0)
    for j in range(CHW - 4, CHW):
        b = j % NSLOT
        _gwait(j, b)
        _scat(j, b, j)
    for b in range(NSLOT):
        _swait(0, b, b)
    plsc.subcore_barrier()
    pltpu.sync_copy(acc_e.at[pl.ds(sid * RPS, RPS)],
                    out_hbm.at[cid * 2, pl.ds(sid * RPS, RPS)])
    pltpu.sync_copy(acc_o.at[pl.ds(sid * RPS, RPS)],
                    out_hbm.at[cid * 2 + 1, pl.ds(sid * RPS, RPS)])

  return _sc_scatter


_sc_scatter_bf16 = _make_sc_scatter(jnp.bfloat16)


@functools.partial(
    pl.kernel,
    out_type=jax.ShapeDtypeStruct((2, NP, H), jnp.float32),
    mesh=_mesh,
    scratch_types=[
        pltpu.VMEM((CHW, C), jnp.int32),
        pltpu.VMEM((CHW, C), jnp.int32),
        pltpu.VMEM((NSLOT, C, H), jnp.float32),
        pltpu.VMEM_SHARED((NP, H), jnp.float32),
        [pltpu.SemaphoreType.DMA] * NSLOT,
        [pltpu.SemaphoreType.DMA] * NSLOT,
    ],
    compiler_params=_sc_params,
)
def _sc_scatter_f32(h_hbm, src_hbm, dst_hbm, zeros_hbm, out_hbm,
                    src_v, dst_v, rows_v, acc, gsems, ssems):
    cid = lax.axis_index("c")
    sid = lax.axis_index("s")
    wid = sid * 2 + cid
    pltpu.sync_copy(zeros_hbm.at[pl.ds(sid * RPS, RPS)],
                    acc.at[pl.ds(sid * RPS, RPS)])
    pltpu.sync_copy(src_hbm.at[wid], src_v)
    pltpu.sync_copy(dst_hbm.at[wid], dst_v)
    plsc.subcore_barrier()

    def _gather(j, b):
        pltpu.async_copy(h_hbm.at[src_v.at[j]], rows_v.at[b], gsems[b])

    def _gwait(j, b):
        pltpu.make_async_copy(h_hbm.at[src_v.at[j]], rows_v.at[b],
                              gsems[b]).wait()

    def _scat(j, b):
        pltpu.async_copy(rows_v.at[b], acc.at[dst_v.at[j]], ssems[b],
                         add=True)

    def _swait(j, b):
        pltpu.make_async_copy(rows_v.at[b], acc.at[dst_v.at[j]],
                              ssems[b]).wait()

    for b in range(4):
        _gather(b, b)
    for j in range(4):
        _gwait(j, j)
        _scat(j, j)
        _gather(j + 4, j + 4)

    def body(g, carry):
        for boff in range(8):
            j = 4 + g * 8 + boff
            b = (4 + boff) % NSLOT
            b4 = boff % NSLOT
            _gwait(j, b)
            _scat(j, b)
            _swait(j - 4, b4)
            _gather(j + 4, b4)
        return carry

    lax.fori_loop(0, (CHW - 8) // 8, body, 0)
    for j in range(CHW - 4, CHW):
        b = j % NSLOT
        _gwait(j, b)
        _scat(j, b)
    for b in range(NSLOT):
        _swait(0, b)
    plsc.subcore_barrier()
    pltpu.sync_copy(acc.at[pl.ds(sid * RPS, RPS)],
                    out_hbm.at[cid, pl.ds(sid * RPS, RPS)])


RB = 1000
_NB = N // RB


def _tc_a_body(x_ref, w1_ref, degp_ref, h_ref, dinv_ref):
    deg = degp_ref[0] + degp_ref[1] + 1.0
    dinv = lax.rsqrt(deg)
    dinv_ref[...] = dinv
    h = jnp.dot(x_ref[...], w1_ref[...], preferred_element_type=jnp.float32)
    h_ref[...] = (h * dinv[:, :1]).astype(jnp.bfloat16)


def _tc_b_body(raw_ref, h1_ref, dinv_ref, b1_ref, w2_ref, h2_ref):
    dinv = dinv_ref[:, :1]
    raw = ((raw_ref[0].astype(jnp.float32) + raw_ref[1].astype(jnp.float32))
           + (raw_ref[2].astype(jnp.float32) + raw_ref[3].astype(jnp.float32)))
    z1 = jnp.maximum((raw + h1_ref[...].astype(jnp.float32)) * dinv
                     + b1_ref[...], 0.0)
    h2 = jnp.dot(z1, w2_ref[...], preferred_element_type=jnp.float32)
    h2_ref[...] = h2 * dinv


def _tc_c_body(raw_ref, h2_ref, dinv_ref, b2_ref, batch_ref,
               wf1_ref, bf1_ref, wf2_ref, bf2_ref, out_ref, accs_ref):
    i = pl.program_id(0)
    dinv = dinv_ref[:, :1]
    raw = raw_ref[0] + raw_ref[1]
    z2 = jnp.maximum((raw + h2_ref[...].astype(jnp.float32)) * dinv
                     + b2_ref[...], 0.0)
    gids = lax.broadcasted_iota(jnp.int32, (RB, G), 1)
    oh = (gids == batch_ref[...]).astype(jnp.float32)
    z2a = jnp.concatenate([z2, jnp.ones((RB, 1), jnp.float32)], 1)
    part = lax.dot_general(oh, z2a, (((0,), (0,)), ((), ())),
                           preferred_element_type=jnp.float32)

    @pl.when(i == 0)
    def _():
        accs_ref[...] = jnp.zeros((G, H + 1), jnp.float32)

    accs_ref[...] += part

    @pl.when(i == _NB - 1)
    def _():
        acc = accs_ref[...]
        pooled = acc[:, :H] / jnp.maximum(acc[:, H:H + 1], 1.0)
        hfc = jnp.maximum(
            jnp.dot(pooled, wf1_ref[...], preferred_element_type=jnp.float32)
            + bf1_ref[...], 0.0)
        out_ref[...] = (jnp.dot(hfc, wf2_ref[...],
                                preferred_element_type=jnp.float32)
                        + bf2_ref[...])


_tc_a = pl.pallas_call(
    _tc_a_body,
    grid=(_NB,),
    in_specs=[
        pl.BlockSpec((RB, D), lambda i: (i, 0)),
        pl.BlockSpec((D, H), lambda i: (0, 0)),
        pl.BlockSpec((2, RB, DEGW), lambda i: (0, i, 0)),
    ],
    out_specs=(pl.BlockSpec((RB, H), lambda i: (i, 0)),
               pl.BlockSpec((RB, DEGW), lambda i: (i, 0))),
    out_shape=(jax.ShapeDtypeStruct((N, H), jnp.bfloat16),
               jax.ShapeDtypeStruct((N, DEGW), jnp.float32)),
)

_tc_b = pl.pallas_call(
    _tc_b_body,
    grid=(_NB,),
    in_specs=[
        pl.BlockSpec((4, RB, H), lambda i: (0, i, 0)),
        pl.BlockSpec((RB, H), lambda i: (i, 0)),
        pl.BlockSpec((RB, DEGW), lambda i: (i, 0)),
        pl.BlockSpec((1, H), lambda i: (0, 0)),
        pl.BlockSpec((H, H), lambda i: (0, 0)),
    ],
    out_specs=pl.BlockSpec((RB, H), lambda i: (i, 0)),
    out_shape=jax.ShapeDtypeStruct((N, H), jnp.float32),
)

_tc_c = pl.pallas_call(
    _tc_c_body,
    grid=(_NB,),
    in_specs=[
        pl.BlockSpec((2, RB, H), lambda i: (0, i, 0)),
        pl.BlockSpec((RB, H), lambda i: (i, 0)),
        pl.BlockSpec((RB, DEGW), lambda i: (i, 0)),
        pl.BlockSpec((1, H), lambda i: (0, 0)),
        pl.BlockSpec((RB, 1), lambda i: (i, 0)),
        pl.BlockSpec((H, H // 2), lambda i: (0, 0)),
        pl.BlockSpec((1, H // 2), lambda i: (0, 0)),
        pl.BlockSpec((H // 2, OUT), lambda i: (0, 0)),
        pl.BlockSpec((1, OUT), lambda i: (0, 0)),
    ],
    out_specs=pl.BlockSpec((G, OUT), lambda i: (0, 0)),
    out_shape=jax.ShapeDtypeStruct((G, OUT), jnp.float32),
    scratch_shapes=[pltpu.VMEM((G, H + 1), jnp.float32)],
)


def kernel(x, edge_index, batch, W1, b1, W2, b2, Wf1, bf1, Wf2, bf2):
    src = edge_index[0].reshape(NW, CHW, C)
    dst = edge_index[1].reshape(NW, CHW, C)
    zeros_b = jnp.zeros((NP, H), jnp.bfloat16)
    zeros_f = jnp.zeros((NP, H), jnp.float32)
    zeros_d = jnp.zeros((NP, DEGW), jnp.float32)
    ones_d = jnp.ones((C, DEGW), jnp.float32)

    degp = _sc_degree(dst, ones_d, zeros_d)
    h1s, dinv = _tc_a(x, W1, degp)
    raw1 = _sc_scatter_bf16(h1s, src, dst, zeros_b)
    h2s = _tc_b(raw1, h1s, dinv, b1.reshape(1, H), W2)
    raw2 = _sc_scatter_f32(h2s, src, dst, zeros_f)
    out = _tc_c(raw2, h2s, dinv, b2.reshape(1, H), batch.reshape(N, 1),
                Wf1, bf1.reshape(1, H // 2), Wf2, bf2.reshape(1, OUT))
    return out


# speedup vs baseline: 1.0102x; 1.0102x over previous
import functools

import jax
import jax.numpy as jnp
from jax import lax
from jax.experimental import pallas as pl
from jax.experimental.pallas import tpu as pltpu
from jax.experimental.pallas import tpu_sc as plsc

N = 10000
E = 320000
D = 128
H = 64
G = 64
OUT = 2

C = 125
NW = 32
CHW = E // (NW * C)
NP = 10240
RPS = NP // 16
DEGW = 16
NSLOT = 8

_mesh = plsc.VectorSubcoreMesh(core_axis_name="c", subcore_axis_name="s")
_sc_params = pltpu.CompilerParams(use_tc_tiling_on_sc=False)


@functools.partial(
    pl.kernel,
    out_type=jax.ShapeDtypeStruct((2, NP, DEGW), jnp.float32),
    mesh=_mesh,
    scratch_types=[
        pltpu.VMEM((CHW, C), jnp.int32),
        pltpu.VMEM((C, DEGW), jnp.float32),
        pltpu.VMEM_SHARED((NP, DEGW), jnp.float32),
        [pltpu.SemaphoreType.DMA] * 4,
    ],
    compiler_params=_sc_params,
)
def _sc_degree(dst_hbm, ones_hbm, zeros_hbm, out_hbm, dst_v, ones_v, acc, sems):
    cid = lax.axis_index("c")
    sid = lax.axis_index("s")
    wid = sid * 2 + cid
    pltpu.sync_copy(zeros_hbm.at[pl.ds(sid * RPS, RPS)],
                    acc.at[pl.ds(sid * RPS, RPS)])
    pltpu.sync_copy(dst_hbm.at[wid], dst_v)
    pltpu.sync_copy(ones_hbm, ones_v)
    plsc.subcore_barrier()

    def _scat(j, b):
        pltpu.async_copy(ones_v, acc.at[dst_v.at[j]], sems[b], add=True)

    def _drain(j, b):
        pltpu.make_async_copy(ones_v, acc.at[dst_v.at[j]], sems[b]).wait()

    for b in range(4):
        _scat(b, b)

    def body(g, carry):
        for b in range(4):
            j = 4 + g * 4 + b
            _drain(j, b)
            _scat(j, b)
        return carry

    lax.fori_loop(0, (CHW - 4) // 4, body, 0)
    for b in range(4):
        _drain(0, b)
    plsc.subcore_barrier()
    pltpu.sync_copy(acc.at[pl.ds(sid * RPS, RPS)],
                    out_hbm.at[cid, pl.ds(sid * RPS, RPS)])


def _make_sc_scatter(dtype):
  @functools.partial(
      pl.kernel,
      out_type=jax.ShapeDtypeStruct((4, NP, H), dtype),
      mesh=_mesh,
      scratch_types=[
          pltpu.VMEM((CHW, C), jnp.int32),
          pltpu.VMEM((CHW, C), jnp.int32),
          pltpu.VMEM((NSLOT, C, H), dtype),
          pltpu.VMEM_SHARED((NP, H), dtype),
          pltpu.VMEM_SHARED((NP, H), dtype),
          [pltpu.SemaphoreType.DMA] * NSLOT,
          [pltpu.SemaphoreType.DMA] * NSLOT,
      ],
      compiler_params=_sc_params,
  )
  def _sc_scatter(h_hbm, src_hbm, dst_hbm, zeros_hbm, out_hbm,
                  src_v, dst_v, rows_v, acc_e, acc_o, gsems, ssems):
    cid = lax.axis_index("c")
    sid = lax.axis_index("s")
    wid = sid * 2 + cid
    pltpu.sync_copy(zeros_hbm.at[pl.ds(sid * RPS, RPS)],
                    acc_e.at[pl.ds(sid * RPS, RPS)])
    pltpu.sync_copy(zeros_hbm.at[pl.ds(sid * RPS, RPS)],
                    acc_o.at[pl.ds(sid * RPS, RPS)])
    pltpu.sync_copy(src_hbm.at[wid], src_v)
    pltpu.sync_copy(dst_hbm.at[wid], dst_v)
    plsc.subcore_barrier()

    def _acc(j):
        return acc_e if j % 2 == 0 else acc_o

    def _gather(j, b):
        pltpu.async_copy(h_hbm.at[src_v.at[j]], rows_v.at[b], gsems[b])

    def _gwait(j, b):
        pltpu.make_async_copy(h_hbm.at[src_v.at[j]], rows_v.at[b],
                              gsems[b]).wait()

    def _scat(j, b, par):
        pltpu.async_copy(rows_v.at[b], _acc(par).at[dst_v.at[j]], ssems[b],
                         add=True)

    def _swait(j, b, par):
        pltpu.make_async_copy(rows_v.at[b], _acc(par).at[dst_v.at[j]],
                              ssems[b]).wait()

    for b in range(4):
        _gather(b, b)
    for j in range(4):
        _gwait(j, j)
        _scat(j, j, j)
        _gather(j + 4, j + 4)

    def body(g, carry):
        for boff in range(8):
            j = 4 + g * 8 + boff
            b = (4 + boff) % NSLOT
            b4 = boff % NSLOT
            _gwait(j, b)
            _scat(j, b, boff)
            _swait(j - 4, b4, boff)
            _gather(j + 4, b4)
        return carry

    lax.fori_loop(0, (CHW - 8) // 8, body, 0)
    for j in range(CHW - 4, CHW):
        b = j % NSLOT
        _gwait(j, b)
        _scat(j, b, j)
    for b in range(NSLOT):
        _swait(0, b, b)
    plsc.subcore_barrier()
    pltpu.sync_copy(acc_e.at[pl.ds(sid * RPS, RPS)],
                    out_hbm.at[cid * 2, pl.ds(sid * RPS, RPS)])
    pltpu.sync_copy(acc_o.at[pl.ds(sid * RPS, RPS)],
                    out_hbm.at[cid * 2 + 1, pl.ds(sid * RPS, RPS)])

  return _sc_scatter


_sc_scatter_bf16 = _make_sc_scatter(jnp.bfloat16)


@functools.partial(
    pl.kernel,
    out_type=jax.ShapeDtypeStruct((2, NP, H), jnp.float32),
    mesh=_mesh,
    scratch_types=[
        pltpu.VMEM((CHW, C), jnp.int32),
        pltpu.VMEM((CHW, C), jnp.int32),
        pltpu.VMEM((NSLOT, C, H), jnp.float32),
        pltpu.VMEM_SHARED((NP, H), jnp.float32),
        [pltpu.SemaphoreType.DMA] * NSLOT,
        [pltpu.SemaphoreType.DMA] * NSLOT,
    ],
    compiler_params=_sc_params,
)
def _sc_scatter_f32(h_hbm, src_hbm, dst_hbm, zeros_hbm, out_hbm,
                    src_v, dst_v, rows_v, acc, gsems, ssems):
    cid = lax.axis_index("c")
    sid = lax.axis_index("s")
    wid = sid * 2 + cid
    pltpu.sync_copy(zeros_hbm.at[pl.ds(sid * RPS, RPS)],
                    acc.at[pl.ds(sid * RPS, RPS)])
    pltpu.sync_copy(src_hbm.at[wid], src_v)
    pltpu.sync_copy(dst_hbm.at[wid], dst_v)
    plsc.subcore_barrier()

    def _gather(j, b):
        pltpu.async_copy(h_hbm.at[src_v.at[j]], rows_v.at[b], gsems[b])

    def _gwait(j, b):
        pltpu.make_async_copy(h_hbm.at[src_v.at[j]], rows_v.at[b],
                              gsems[b]).wait()

    def _scat(j, b):
        pltpu.async_copy(rows_v.at[b], acc.at[dst_v.at[j]], ssems[b],
                         add=True)

    def _swait(j, b):
        pltpu.make_async_copy(rows_v.at[b], acc.at[dst_v.at[j]],
                              ssems[b]).wait()

    for b in range(4):
        _gather(b, b)
    for j in range(4):
        _gwait(j, j)
        _scat(j, j)
        _gather(j + 4, j + 4)

    def body(g, carry):
        for boff in range(8):
            j = 4 + g * 8 + boff
            b = (4 + boff) % NSLOT
            b4 = boff % NSLOT
            _gwait(j, b)
            _scat(j, b)
            _swait(j - 4, b4)
            _gather(j + 4, b4)
        return carry

    lax.fori_loop(0, (CHW - 8) // 8, body, 0)
    for j in range(CHW - 4, CHW):
        b = j % NSLOT
        _gwait(j, b)
        _scat(j, b)
    for b in range(NSLOT):
        _swait(0, b)
    plsc.subcore_barrier()
    pltpu.sync_copy(acc.at[pl.ds(sid * RPS, RPS)],
                    out_hbm.at[cid, pl.ds(sid * RPS, RPS)])


RB = 1000
_NB = N // RB


def _tc_a_body(x_ref, w1_ref, degp_ref, h_ref, dinv_ref):
    deg = degp_ref[0] + degp_ref[1] + 1.0
    dinv = lax.rsqrt(deg)
    dinv_ref[...] = dinv
    h = jnp.dot(x_ref[...], w1_ref[...], preferred_element_type=jnp.float32)
    h_ref[...] = (h * dinv[:, :1]).astype(jnp.bfloat16)


def _tc_b_body(raw_ref, h1_ref, dinv_ref, b1_ref, w2_ref, h2_ref):
    dinv = dinv_ref[:, :1]
    raw = ((raw_ref[0].astype(jnp.float32) + raw_ref[1].astype(jnp.float32))
           + (raw_ref[2].astype(jnp.float32) + raw_ref[3].astype(jnp.float32)))
    z1 = jnp.maximum((raw + h1_ref[...].astype(jnp.float32)) * dinv
                     + b1_ref[...], 0.0)
    h2 = jnp.dot(z1, w2_ref[...], preferred_element_type=jnp.float32)
    h2_ref[...] = (h2 * dinv).astype(jnp.bfloat16)


def _tc_c_body(raw_ref, h2_ref, dinv_ref, b2_ref, batch_ref,
               wf1_ref, bf1_ref, wf2_ref, bf2_ref, out_ref, accs_ref):
    i = pl.program_id(0)
    dinv = dinv_ref[:, :1]
    raw = ((raw_ref[0].astype(jnp.float32) + raw_ref[1].astype(jnp.float32))
           + (raw_ref[2].astype(jnp.float32) + raw_ref[3].astype(jnp.float32)))
    z2 = jnp.maximum((raw + h2_ref[...].astype(jnp.float32)) * dinv
                     + b2_ref[...], 0.0)
    gids = lax.broadcasted_iota(jnp.int32, (RB, G), 1)
    oh = (gids == batch_ref[...]).astype(jnp.float32)
    z2a = jnp.concatenate([z2, jnp.ones((RB, 1), jnp.float32)], 1)
    part = lax.dot_general(oh, z2a, (((0,), (0,)), ((), ())),
                           preferred_element_type=jnp.float32)

    @pl.when(i == 0)
    def _():
        accs_ref[...] = jnp.zeros((G, H + 1), jnp.float32)

    accs_ref[...] += part

    @pl.when(i == _NB - 1)
    def _():
        acc = accs_ref[...]
        pooled = acc[:, :H] / jnp.maximum(acc[:, H:H + 1], 1.0)
        hfc = jnp.maximum(
            jnp.dot(pooled, wf1_ref[...], preferred_element_type=jnp.float32)
            + bf1_ref[...], 0.0)
        out_ref[...] = (jnp.dot(hfc, wf2_ref[...],
                                preferred_element_type=jnp.float32)
                        + bf2_ref[...])


_tc_a = pl.pallas_call(
    _tc_a_body,
    grid=(_NB,),
    in_specs=[
        pl.BlockSpec((RB, D), lambda i: (i, 0)),
        pl.BlockSpec((D, H), lambda i: (0, 0)),
        pl.BlockSpec((2, RB, DEGW), lambda i: (0, i, 0)),
    ],
    out_specs=(pl.BlockSpec((RB, H), lambda i: (i, 0)),
               pl.BlockSpec((RB, DEGW), lambda i: (i, 0))),
    out_shape=(jax.ShapeDtypeStruct((N, H), jnp.bfloat16),
               jax.ShapeDtypeStruct((N, DEGW), jnp.float32)),
)

_tc_b = pl.pallas_call(
    _tc_b_body,
    grid=(_NB,),
    in_specs=[
        pl.BlockSpec((4, RB, H), lambda i: (0, i, 0)),
        pl.BlockSpec((RB, H), lambda i: (i, 0)),
        pl.BlockSpec((RB, DEGW), lambda i: (i, 0)),
        pl.BlockSpec((1, H), lambda i: (0, 0)),
        pl.BlockSpec((H, H), lambda i: (0, 0)),
    ],
    out_specs=pl.BlockSpec((RB, H), lambda i: (i, 0)),
    out_shape=jax.ShapeDtypeStruct((N, H), jnp.bfloat16),
)

_tc_c = pl.pallas_call(
    _tc_c_body,
    grid=(_NB,),
    in_specs=[
        pl.BlockSpec((4, RB, H), lambda i: (0, i, 0)),
        pl.BlockSpec((RB, H), lambda i: (i, 0)),
        pl.BlockSpec((RB, DEGW), lambda i: (i, 0)),
        pl.BlockSpec((1, H), lambda i: (0, 0)),
        pl.BlockSpec((RB, 1), lambda i: (i, 0)),
        pl.BlockSpec((H, H // 2), lambda i: (0, 0)),
        pl.BlockSpec((1, H // 2), lambda i: (0, 0)),
        pl.BlockSpec((H // 2, OUT), lambda i: (0, 0)),
        pl.BlockSpec((1, OUT), lambda i: (0, 0)),
    ],
    out_specs=pl.BlockSpec((G, OUT), lambda i: (0, 0)),
    out_shape=jax.ShapeDtypeStruct((G, OUT), jnp.float32),
    scratch_shapes=[pltpu.VMEM((G, H + 1), jnp.float32)],
)


def kernel(x, edge_index, batch, W1, b1, W2, b2, Wf1, bf1, Wf2, bf2):
    src = edge_index[0].reshape(NW, CHW, C)
    dst = edge_index[1].reshape(NW, CHW, C)
    zeros_b = jnp.zeros((NP, H), jnp.bfloat16)
    zeros_f = jnp.zeros((NP, H), jnp.float32)
    zeros_d = jnp.zeros((NP, DEGW), jnp.float32)
    ones_d = jnp.ones((C, DEGW), jnp.float32)

    degp = _sc_degree(dst, ones_d, zeros_d)
    h1s, dinv = _tc_a(x, W1, degp)
    raw1 = _sc_scatter_bf16(h1s, src, dst, zeros_b)
    h2s = _tc_b(raw1, h1s, dinv, b1.reshape(1, H), W2)
    raw2 = _sc_scatter_bf16(h2s, src, dst, zeros_b)
    out = _tc_c(raw2, h2s, dinv, b2.reshape(1, H), batch.reshape(N, 1),
                Wf1, bf1.reshape(1, H // 2), Wf2, bf2.reshape(1, OUT))
    return out
